# Initial kernel scaffold; baseline (speedup 1.0000x reference)
import jax
import jax.numpy as jnp
from jax.experimental import pallas as pl


def kernel(scores):
    raise NotImplementedError("write your pallas kernel here")


# speedup vs baseline: 11.7544x; 11.7544x over previous
import jax
import jax.numpy as jnp
from jax.experimental import pallas as pl
from jax.experimental.pallas import tpu as pltpu

_K_TOP = 512
_EPS = 0.1
_MAX_ITER = 200
_ROWS = 64


def _stk_kernel(s_ref, o_ref, e0_ref, e1_ref, e2_ref):
    s = s_ref[...]
    smin = jnp.min(s, axis=1, keepdims=True)
    smax = jnp.max(s, axis=1, keepdims=True)
    x = (s - smin) / (smax - smin + jnp.float32(1e-12))

    inv_eps = jnp.float32(1.0 / _EPS)
    e0_ref[...] = jnp.exp(-(x * x) * inv_eps)
    xm = x - jnp.float32(0.5)
    e1_ref[...] = jnp.exp(-(xm * xm) * inv_eps)
    xm = x - jnp.float32(1.0)
    e2_ref[...] = jnp.exp(-(xm * xm) * inv_eps)

    n = s.shape[1]
    nu0n = jnp.float32(_K_TOP)
    nu1n = jnp.float32(n - 2 * _K_TOP)
    nu2n = jnp.float32(_K_TOP)

    def half_step(a0, a1, a2):
        e0 = e0_ref[...]
        e1 = e1_ref[...]
        e2 = e2_ref[...]
        s_plane = e0 * a0 + e1 * a1 + e2 * a2
        inv_s = jnp.float32(1.0) / s_plane
        r0 = jnp.sum(e0 * inv_s, axis=1, keepdims=True)
        r1 = jnp.sum(e1 * inv_s, axis=1, keepdims=True)
        r2 = jnp.sum(e2 * inv_s, axis=1, keepdims=True)
        return inv_s, r0, r1, r2

    def body(_, carry):
        a0, a1, a2 = carry
        _, r0, r1, r2 = half_step(a0, a1, a2)
        return (nu0n / r0, nu1n / r1, nu2n / r2)

    ones = jnp.ones((s.shape[0], 1), jnp.float32)
    a0, a1, a2 = jax.lax.fori_loop(0, _MAX_ITER - 1, body, (ones, ones, ones))

    inv_s, r0, _, r2 = half_step(a0, a1, a2)
    a0 = nu0n / r0
    a2 = nu2n / r2
    o_ref[...] = (e2_ref[...] * a2 - e0_ref[...] * a0) * inv_s


@jax.jit
def kernel(scores):
    b, n = scores.shape
    return pl.pallas_call(
        _stk_kernel,
        grid=(b // _ROWS,),
        in_specs=[pl.BlockSpec((_ROWS, n), lambda i: (i, 0))],
        out_specs=pl.BlockSpec((_ROWS, n), lambda i: (i, 0)),
        out_shape=jax.ShapeDtypeStruct((b, n), jnp.float32),
        scratch_shapes=[
            pltpu.VMEM((_ROWS, n), jnp.float32),
            pltpu.VMEM((_ROWS, n), jnp.float32),
            pltpu.VMEM((_ROWS, n), jnp.float32),
        ],
        compiler_params=pltpu.CompilerParams(
            dimension_semantics=("parallel",),
        ),
    )(scores)


# speedup vs baseline: 14.0266x; 1.1933x over previous
import jax
import jax.numpy as jnp
from jax.experimental import pallas as pl
from jax.experimental.pallas import tpu as pltpu

_K_TOP = 512
_EPS = 0.1
_MAX_ITER = 200
_ROWS = 64


def _stk_kernel(s_ref, o_ref, q0_ref, q2_ref):
    s = s_ref[...]
    smin = jnp.min(s, axis=1, keepdims=True)
    smax = jnp.max(s, axis=1, keepdims=True)
    x = (s - smin) / (smax - smin + jnp.float32(1e-12))

    inv_eps = jnp.float32(1.0 / _EPS)
    q0_ref[...] = jnp.exp((jnp.float32(0.25) - x) * inv_eps)
    q2_ref[...] = jnp.exp((x - jnp.float32(0.75)) * inv_eps)

    n = s.shape[1]
    nu0n = jnp.float32(_K_TOP)
    nu1n = jnp.float32(n - 2 * _K_TOP)
    nu2n = jnp.float32(_K_TOP)

    def half_step(a0, a1, a2):
        q0 = q0_ref[...]
        q2 = q2_ref[...]
        inv_t = jnp.float32(1.0) / (a1 + q0 * a0 + q2 * a2)
        r0 = jnp.sum(q0 * inv_t, axis=1, keepdims=True)
        r1 = jnp.sum(inv_t, axis=1, keepdims=True)
        r2 = jnp.sum(q2 * inv_t, axis=1, keepdims=True)
        return inv_t, r0, r1, r2

    def body(_, carry):
        a0, a1, a2 = carry
        _, r0, r1, r2 = half_step(a0, a1, a2)
        return (nu0n / r0, nu1n / r1, nu2n / r2)

    ones = jnp.ones((s.shape[0], 1), jnp.float32)
    a0, a1, a2 = jax.lax.fori_loop(0, _MAX_ITER - 1, body, (ones, ones, ones))

    inv_t, r0, _, r2 = half_step(a0, a1, a2)
    a0 = nu0n / r0
    a2 = nu2n / r2
    o_ref[...] = (q2_ref[...] * a2 - q0_ref[...] * a0) * inv_t


@jax.jit
def kernel(scores):
    b, n = scores.shape
    return pl.pallas_call(
        _stk_kernel,
        grid=(b // _ROWS,),
        in_specs=[pl.BlockSpec((_ROWS, n), lambda i: (i, 0))],
        out_specs=pl.BlockSpec((_ROWS, n), lambda i: (i, 0)),
        out_shape=jax.ShapeDtypeStruct((b, n), jnp.float32),
        scratch_shapes=[
            pltpu.VMEM((_ROWS, n), jnp.float32),
            pltpu.VMEM((_ROWS, n), jnp.float32),
        ],
        compiler_params=pltpu.CompilerParams(
            dimension_semantics=("parallel",),
        ),
    )(scores)


# speedup vs baseline: 17.6617x; 1.2592x over previous
import jax
import jax.numpy as jnp
from jax.experimental import pallas as pl
from jax.experimental.pallas import tpu as pltpu

_K_TOP = 512
_EPS = 0.1
_MAX_ITER = 200
_ROWS = 64
_CHUNK = 512


def _stk_kernel(s_ref, o_ref, q0_ref, q2_ref):
    s = s_ref[...]
    smin = jnp.min(s, axis=1, keepdims=True)
    smax = jnp.max(s, axis=1, keepdims=True)
    x = (s - smin) / (smax - smin + jnp.float32(1e-12))

    inv_eps = jnp.float32(1.0 / _EPS)
    q0_ref[...] = jnp.exp((jnp.float32(0.25) - x) * inv_eps)
    q2_ref[...] = jnp.exp((x - jnp.float32(0.75)) * inv_eps)

    rows, n = s.shape
    n_chunks = n // _CHUNK
    nf = jnp.float32(n)
    nu0n = jnp.float32(_K_TOP)
    nu1n = jnp.float32(n - 2 * _K_TOP)
    nu2n = jnp.float32(_K_TOP)
    one = jnp.float32(1.0)

    def column_sums(a0, a1, a2):
        acc1 = jnp.zeros((rows, _CHUNK), jnp.float32)
        acc2 = jnp.zeros((rows, _CHUNK), jnp.float32)
        for c in range(n_chunks):
            sl = slice(c * _CHUNK, (c + 1) * _CHUNK)
            q0c = q0_ref[:, sl]
            q2c = q2_ref[:, sl]
            w = one / (a1 + q0c * a0 + q2c * a2)
            acc1 = acc1 + w
            acc2 = acc2 + q2c * w
        r1 = jnp.sum(acc1, axis=1, keepdims=True)
        r2 = jnp.sum(acc2, axis=1, keepdims=True)
        return r1, r2

    def new_scales(a0, a1, a2, r1, r2):
        a0n = nu0n * a0 / (nf - a1 * r1 - a2 * r2)
        return a0n, nu1n / r1, nu2n / r2

    def body(_, carry):
        a0, a1, a2 = carry
        r1, r2 = column_sums(a0, a1, a2)
        return new_scales(a0, a1, a2, r1, r2)

    ones = jnp.ones((rows, 1), jnp.float32)
    a0, a1, a2 = jax.lax.fori_loop(0, _MAX_ITER - 1, body, (ones, ones, ones))

    r1, r2 = column_sums(a0, a1, a2)
    a0n, _, a2n = new_scales(a0, a1, a2, r1, r2)
    for c in range(n_chunks):
        sl = slice(c * _CHUNK, (c + 1) * _CHUNK)
        q0c = q0_ref[:, sl]
        q2c = q2_ref[:, sl]
        w = one / (a1 + q0c * a0 + q2c * a2)
        o_ref[:, sl] = (q2c * a2n - q0c * a0n) * w


@jax.jit
def kernel(scores):
    b, n = scores.shape
    return pl.pallas_call(
        _stk_kernel,
        grid=(b // _ROWS,),
        in_specs=[pl.BlockSpec((_ROWS, n), lambda i: (i, 0))],
        out_specs=pl.BlockSpec((_ROWS, n), lambda i: (i, 0)),
        out_shape=jax.ShapeDtypeStruct((b, n), jnp.float32),
        scratch_shapes=[
            pltpu.VMEM((_ROWS, n), jnp.float32),
            pltpu.VMEM((_ROWS, n), jnp.float32),
        ],
        compiler_params=pltpu.CompilerParams(
            dimension_semantics=("parallel",),
        ),
    )(scores)
